# baseline (speedup 1.0000x reference)
import functools

import jax
import jax.numpy as jnp
from jax import lax
from jax.experimental import pallas as pl
from jax.experimental.pallas import tpu as pltpu
from jax.experimental.pallas import tpu_sc as plsc

N = 10000
E = 320000
D_H = 16

NW = 32
CHUNK = 128
ROWS = E // CHUNK
CPW = ROWS // NW
EXTRA = ROWS - NW * CPW
NPAD = 10112
RPT = NPAD // 16
NITER = CPW // 2


def _edge_pass_body(with_deg, *refs):
    if with_deg:
        (zeros_hbm, ones_hbm, table_hbm, src_hbm, dst_hbm,
         agg_out, deg_out,
         si, di, buf_a, buf_b, ones_v, agg_s, deg_s,
         gsem_a, gsem_b, ssem_a, ssem_b, dsem_a, dsem_b) = refs
    else:
        (zeros_hbm, table_hbm, src_hbm, dst_hbm,
         agg_out,
         si, di, buf_a, buf_b, agg_s,
         gsem_a, gsem_b, ssem_a, ssem_b) = refs

    cid = lax.axis_index("c")
    sid = lax.axis_index("s")
    w = cid * 16 + sid

    pltpu.sync_copy(src_hbm.at[pl.ds(w * CPW, CPW)], si.at[pl.ds(0, CPW)])
    pltpu.sync_copy(dst_hbm.at[pl.ds(w * CPW, CPW)], di.at[pl.ds(0, CPW)])

    @pl.when(w < EXTRA)
    def _():
        pltpu.sync_copy(src_hbm.at[pl.ds(NW * CPW + w, 1)],
                        si.at[pl.ds(CPW, 1)])
        pltpu.sync_copy(dst_hbm.at[pl.ds(NW * CPW + w, 1)],
                        di.at[pl.ds(CPW, 1)])

    z = sid * RPT
    pltpu.sync_copy(zeros_hbm.at[pl.ds(z, RPT)], agg_s.at[pl.ds(z, RPT)])
    if with_deg:
        pltpu.sync_copy(zeros_hbm.at[pl.ds(z, RPT)], deg_s.at[pl.ds(z, RPT)])
        pltpu.sync_copy(ones_hbm, ones_v)
    plsc.subcore_barrier()

    def gather(j, buf, sem):
        return pltpu.make_async_copy(table_hbm.at[si.at[j]], buf, sem)

    def scatter(j, buf, sem):
        return pltpu.make_async_copy(buf, agg_s.at[di.at[j]], sem)

    def deg_scatter(j, sem):
        return pltpu.make_async_copy(ones_v, deg_s.at[di.at[j]], sem)

    def scatter_start(j, buf, sem, dsem):
        scatter(j, buf, sem).start(add=True)
        if with_deg:
            deg_scatter(j, dsem).start(add=True)

    def scatter_wait(j, buf, sem, dsem):
        scatter(j, buf, sem).wait()
        if with_deg:
            deg_scatter(j, dsem).wait()

    gather(0, buf_a, gsem_a).start()
    gather(1, buf_b, gsem_b).start()

    def loop_body(i, carry):
        j0 = 2 * i
        j1 = j0 + 1
        gather(j0, buf_a, gsem_a).wait()
        scatter_start(j0, buf_a, ssem_a, dsem_a if with_deg else None)
        gather(j1, buf_b, gsem_b).wait()
        scatter_start(j1, buf_b, ssem_b, dsem_b if with_deg else None)

        @pl.when(i < NITER - 1)
        def _():
            scatter_wait(j0, buf_a, ssem_a, dsem_a if with_deg else None)
            gather(j0 + 2, buf_a, gsem_a).start()
            scatter_wait(j1, buf_b, ssem_b, dsem_b if with_deg else None)
            gather(j1 + 2, buf_b, gsem_b).start()

        return carry

    lax.fori_loop(0, NITER, loop_body, 0)
    scatter_wait(2 * NITER - 2, buf_a, ssem_a, dsem_a if with_deg else None)
    scatter_wait(2 * NITER - 1, buf_b, ssem_b, dsem_b if with_deg else None)

    @pl.when(w < EXTRA)
    def _():
        gather(CPW, buf_a, gsem_a).start()
        gather(CPW, buf_a, gsem_a).wait()
        scatter(CPW, buf_a, ssem_a).start(add=True)
        if with_deg:
            deg_scatter(CPW, dsem_a).start(add=True)
        scatter(CPW, buf_a, ssem_a).wait()
        if with_deg:
            deg_scatter(CPW, dsem_a).wait()

    plsc.subcore_barrier()

    pltpu.sync_copy(agg_s.at[pl.ds(z, RPT)], agg_out.at[cid, pl.ds(z, RPT)])
    if with_deg:
        pltpu.sync_copy(deg_s.at[pl.ds(z, RPT)],
                        deg_out.at[cid, pl.ds(z, RPT)])


def _make_edge_pass(with_deg):
    mesh = plsc.VectorSubcoreMesh(core_axis_name="c", subcore_axis_name="s")
    out_type = [jax.ShapeDtypeStruct((2, NPAD, D_H), jnp.float32)]
    if with_deg:
        out_type.append(jax.ShapeDtypeStruct((2, NPAD, D_H), jnp.float32))
    scratch = [
        pltpu.VMEM((CPW + 1, CHUNK), jnp.int32),
        pltpu.VMEM((CPW + 1, CHUNK), jnp.int32),
        pltpu.VMEM((CHUNK, D_H), jnp.float32),
        pltpu.VMEM((CHUNK, D_H), jnp.float32),
    ]
    if with_deg:
        scratch.append(pltpu.VMEM((CHUNK, D_H), jnp.float32))
    scratch.append(pltpu.VMEM_SHARED((NPAD, D_H), jnp.float32))
    if with_deg:
        scratch.append(pltpu.VMEM_SHARED((NPAD, D_H), jnp.float32))
    scratch += [pltpu.SemaphoreType.DMA] * (6 if with_deg else 4)

    return pl.kernel(
        functools.partial(_edge_pass_body, with_deg),
        mesh=mesh,
        out_type=tuple(out_type) if with_deg else out_type[0],
        scratch_types=scratch,
        compiler_params=pltpu.CompilerParams(use_tc_tiling_on_sc=False),
    )


NP = N * D_H // 128
NPP = NPAD * D_H // 128
GROUP = 128 // D_H


def _dense1_body(x_ref, wl_ref, wr_ref, y_ref, r_ref):
    x = x_ref[...]
    i = lax.broadcasted_iota(jnp.int32, (GROUP * 128, 128), 0)
    j = lax.broadcasted_iota(jnp.int32, (GROUP * 128, 128), 1)
    m = (i // 128) == (j // D_H)

    def big(w):
        wcat = jnp.concatenate([w] * GROUP, axis=0)
        wt = jnp.concatenate([wcat] * GROUP, axis=1)
        return jnp.where(m, wt, 0.0)

    y_ref[...] = jnp.dot(x, big(wl_ref[...]),
                         preferred_element_type=jnp.float32)
    r_ref[...] = jnp.dot(x, big(wr_ref[...]),
                         preferred_element_type=jnp.float32)


def _mid_body(aggp_ref, degp_ref, r_ref, b1_ref, h_ref, dinv_ref):
    agg = aggp_ref[0, :NP] + aggp_ref[1, :NP]
    deg = degp_ref[0, :NP] + degp_ref[1, :NP]
    i = lax.broadcasted_iota(jnp.int32, (128, 128), 0)
    j = lax.broadcasted_iota(jnp.int32, (128, 128), 1)
    b = jnp.where((i % D_H == 0) & (i // D_H == j // D_H), 1.0, 0.0)
    degb = jnp.dot(deg, b, preferred_element_type=jnp.float32)
    dinv = 1.0 / jnp.maximum(degb, 1.0)
    b1row = jnp.concatenate([b1_ref[...]] * GROUP, axis=1)
    h_ref[...] = jnp.maximum(agg * dinv + b1row + r_ref[...], 0.0)
    dinv_ref[...] = dinv


def _out_body(agg2p_ref, dinv_ref, h_ref, w2l_ref, w2r_ref, b2_ref,
              fcw_ref, fcb_ref, out_ref):
    h = h_ref[...]
    agg2 = (agg2p_ref[0, :NP] + agg2p_ref[1, :NP]) * dinv_ref[...]
    fcw = fcw_ref[...]
    wl = jnp.dot(w2l_ref[...], fcw, preferred_element_type=jnp.float32)
    wr = jnp.dot(w2r_ref[...], fcw, preferred_element_type=jnp.float32)
    c = jnp.dot(b2_ref[...], fcw, preferred_element_type=jnp.float32) + fcb_ref[...]
    i = lax.broadcasted_iota(jnp.int32, (128, GROUP), 0)
    j = lax.broadcasted_iota(jnp.int32, (128, GROUP), 1)
    wl_cat = jnp.concatenate([wl] * GROUP, axis=0)
    wr_cat = jnp.concatenate([wr] * GROUP, axis=0)
    kl = jnp.where(i // D_H == j, wl_cat, 0.0)
    kr = jnp.where(i // D_H == j, wr_cat, 0.0)
    out_ref[...] = (jnp.dot(agg2, kl, preferred_element_type=jnp.float32)
                    + jnp.dot(h, kr, preferred_element_type=jnp.float32) + c)


def kernel(x, edge_index, W1_l, W1_r, b1, W2_l, W2_r, b2, fc_w, fc_b):
    ei3 = edge_index.reshape(2, ROWS, CHUNK).transpose(1, 0, 2)
    src_r = ei3[:, 0, :]
    dst_r = ei3[:, 1, :]
    zeros = jnp.zeros((NPAD, D_H), jnp.float32)
    onesc = jnp.tile(
        jnp.array([1.0] + [0.0] * (D_H - 1), jnp.float32)[None, :],
        (CHUNK, 1))
    x2 = x.reshape(NP, GROUP * 128)

    y_p, r_p = pl.pallas_call(
        _dense1_body,
        out_shape=(jax.ShapeDtypeStruct((NP, 128), jnp.float32),
                   jax.ShapeDtypeStruct((NP, 128), jnp.float32)),
    )(x2, W1_l, W1_r)

    aggp, degp = _make_edge_pass(True)(
        zeros, onesc, y_p.reshape(N, D_H), src_r, dst_r)

    h_p, dinv_p = pl.pallas_call(
        _mid_body,
        out_shape=(jax.ShapeDtypeStruct((NP, 128), jnp.float32),
                   jax.ShapeDtypeStruct((NP, 128), jnp.float32)),
    )(aggp.reshape(2, NPP, 128), degp.reshape(2, NPP, 128), r_p,
      b1.reshape(1, D_H))

    agg2p = _make_edge_pass(False)(zeros, h_p.reshape(N, D_H), src_r, dst_r)

    out_p = pl.pallas_call(
        _out_body,
        out_shape=jax.ShapeDtypeStruct((NP, GROUP), jnp.float32),
    )(agg2p.reshape(2, NPP, 128), dinv_p, h_p, W2_l, W2_r,
      b2.reshape(1, 8), fc_w, fc_b.reshape(1, 1))
    return out_p.reshape(N, 1)


# speedup vs baseline: 1.3264x; 1.3264x over previous
import functools

import jax
import jax.numpy as jnp
from jax import lax
from jax.experimental import pallas as pl
from jax.experimental.pallas import tpu as pltpu
from jax.experimental.pallas import tpu_sc as plsc

N = 10000
E = 320000
D_H = 16

NW = 32
CHUNK = 128
ROWS = E // CHUNK
CPW = ROWS // NW
EXTRA = ROWS - NW * CPW
NPAD = 10112
RPT = NPAD // 16
NITER = CPW // 2


def _edge_pass_body(with_deg, *refs):
    if with_deg:
        (zeros_hbm, ones_hbm, table_hbm, ei_hbm,
         agg_out, deg_out,
         ei, buf_a, buf_b, ones_v, agg_s, deg_s, table_s,
         gsem_a, gsem_b, ssem_a, ssem_b, dsem_a, dsem_b) = refs
    else:
        (zeros_hbm, table_hbm, ei_hbm,
         agg_out,
         ei, buf_a, buf_b, agg_s, table_s,
         gsem_a, gsem_b, ssem_a, ssem_b) = refs

    cid = lax.axis_index("c")
    sid = lax.axis_index("s")
    w = cid * 16 + sid

    pltpu.sync_copy(ei_hbm.at[pl.ds(w * CPW, CPW)], ei.at[pl.ds(0, CPW)])

    @pl.when(w < EXTRA)
    def _():
        pltpu.sync_copy(ei_hbm.at[pl.ds(NW * CPW + w, 1)],
                        ei.at[pl.ds(CPW, 1)])

    z = sid * RPT
    zt = sid * (N // 16)
    pltpu.sync_copy(zeros_hbm.at[pl.ds(z, RPT)], agg_s.at[pl.ds(z, RPT)])
    pltpu.sync_copy(table_hbm.at[pl.ds(zt, N // 16)],
                    table_s.at[pl.ds(zt, N // 16)])
    if with_deg:
        pltpu.sync_copy(zeros_hbm.at[pl.ds(z, RPT)], deg_s.at[pl.ds(z, RPT)])
        pltpu.sync_copy(ones_hbm, ones_v)
    plsc.subcore_barrier()

    def gather(j, buf, sem):
        return pltpu.make_async_copy(table_s.at[ei.at[j, 0]], buf, sem)

    def scatter(j, buf, sem):
        return pltpu.make_async_copy(buf, agg_s.at[ei.at[j, 1]], sem)

    def deg_scatter(j, sem):
        return pltpu.make_async_copy(ones_v, deg_s.at[ei.at[j, 1]], sem)

    def scatter_start(j, buf, sem, dsem):
        scatter(j, buf, sem).start(add=True)
        if with_deg:
            deg_scatter(j, dsem).start(add=True)

    def scatter_wait(j, buf, sem, dsem):
        scatter(j, buf, sem).wait()
        if with_deg:
            deg_scatter(j, dsem).wait()

    gather(0, buf_a, gsem_a).start()
    gather(1, buf_b, gsem_b).start()

    def loop_body(i, carry):
        j0 = 2 * i
        j1 = j0 + 1
        gather(j0, buf_a, gsem_a).wait()
        scatter_start(j0, buf_a, ssem_a, dsem_a if with_deg else None)
        gather(j1, buf_b, gsem_b).wait()
        scatter_start(j1, buf_b, ssem_b, dsem_b if with_deg else None)

        @pl.when(i < NITER - 1)
        def _():
            scatter_wait(j0, buf_a, ssem_a, dsem_a if with_deg else None)
            gather(j0 + 2, buf_a, gsem_a).start()
            scatter_wait(j1, buf_b, ssem_b, dsem_b if with_deg else None)
            gather(j1 + 2, buf_b, gsem_b).start()

        return carry

    lax.fori_loop(0, NITER, loop_body, 0)
    scatter_wait(2 * NITER - 2, buf_a, ssem_a, dsem_a if with_deg else None)
    scatter_wait(2 * NITER - 1, buf_b, ssem_b, dsem_b if with_deg else None)

    @pl.when(w < EXTRA)
    def _():
        gather(CPW, buf_a, gsem_a).start()
        gather(CPW, buf_a, gsem_a).wait()
        scatter(CPW, buf_a, ssem_a).start(add=True)
        if with_deg:
            deg_scatter(CPW, dsem_a).start(add=True)
        scatter(CPW, buf_a, ssem_a).wait()
        if with_deg:
            deg_scatter(CPW, dsem_a).wait()

    plsc.subcore_barrier()

    pltpu.sync_copy(agg_s.at[pl.ds(z, RPT)], agg_out.at[cid, pl.ds(z, RPT)])
    if with_deg:
        pltpu.sync_copy(deg_s.at[pl.ds(z, RPT)],
                        deg_out.at[cid, pl.ds(z, RPT)])


def _make_edge_pass(with_deg):
    mesh = plsc.VectorSubcoreMesh(core_axis_name="c", subcore_axis_name="s")
    out_type = [jax.ShapeDtypeStruct((2, NPAD, D_H), jnp.float32)]
    if with_deg:
        out_type.append(jax.ShapeDtypeStruct((2, NPAD, D_H), jnp.float32))
    scratch = [
        pltpu.VMEM((CPW + 1, 2, CHUNK), jnp.int32),
        pltpu.VMEM((CHUNK, D_H), jnp.float32),
        pltpu.VMEM((CHUNK, D_H), jnp.float32),
    ]
    if with_deg:
        scratch.append(pltpu.VMEM((CHUNK, D_H), jnp.float32))
    scratch.append(pltpu.VMEM_SHARED((NPAD, D_H), jnp.float32))
    if with_deg:
        scratch.append(pltpu.VMEM_SHARED((NPAD, D_H), jnp.float32))
    scratch.append(pltpu.VMEM_SHARED((N, D_H), jnp.float32))
    scratch += [pltpu.SemaphoreType.DMA] * (6 if with_deg else 4)

    return pl.kernel(
        functools.partial(_edge_pass_body, with_deg),
        mesh=mesh,
        out_type=tuple(out_type) if with_deg else out_type[0],
        scratch_types=scratch,
        compiler_params=pltpu.CompilerParams(use_tc_tiling_on_sc=False),
    )


NP = N * D_H // 128
NPP = NPAD * D_H // 128
GROUP = 128 // D_H


def _dense1_body(x_ref, wl_ref, wr_ref, y_ref, r_ref):
    x = x_ref[...]
    i = lax.broadcasted_iota(jnp.int32, (GROUP * 128, 128), 0)
    j = lax.broadcasted_iota(jnp.int32, (GROUP * 128, 128), 1)
    m = (i // 128) == (j // D_H)

    def big(w):
        wcat = jnp.concatenate([w] * GROUP, axis=0)
        wt = jnp.concatenate([wcat] * GROUP, axis=1)
        return jnp.where(m, wt, 0.0)

    y_ref[...] = jnp.dot(x, big(wl_ref[...]),
                         preferred_element_type=jnp.float32)
    r_ref[...] = jnp.dot(x, big(wr_ref[...]),
                         preferred_element_type=jnp.float32)


def _mid_body(aggp_ref, degp_ref, r_ref, b1_ref, h_ref, dinv_ref):
    agg = aggp_ref[0, :NP] + aggp_ref[1, :NP]
    deg = degp_ref[0, :NP] + degp_ref[1, :NP]
    i = lax.broadcasted_iota(jnp.int32, (128, 128), 0)
    j = lax.broadcasted_iota(jnp.int32, (128, 128), 1)
    b = jnp.where((i % D_H == 0) & (i // D_H == j // D_H), 1.0, 0.0)
    degb = jnp.dot(deg, b, preferred_element_type=jnp.float32)
    dinv = 1.0 / jnp.maximum(degb, 1.0)
    b1row = jnp.concatenate([b1_ref[...]] * GROUP, axis=1)
    h_ref[...] = jnp.maximum(agg * dinv + b1row + r_ref[...], 0.0)
    dinv_ref[...] = dinv


def _out_body(agg2p_ref, dinv_ref, h_ref, w2l_ref, w2r_ref, b2_ref,
              fcw_ref, fcb_ref, out_ref):
    h = h_ref[...]
    agg2 = (agg2p_ref[0, :NP] + agg2p_ref[1, :NP]) * dinv_ref[...]
    fcw = fcw_ref[...]
    wl = jnp.dot(w2l_ref[...], fcw, preferred_element_type=jnp.float32)
    wr = jnp.dot(w2r_ref[...], fcw, preferred_element_type=jnp.float32)
    c = jnp.dot(b2_ref[...], fcw, preferred_element_type=jnp.float32) + fcb_ref[...]
    i = lax.broadcasted_iota(jnp.int32, (128, GROUP), 0)
    j = lax.broadcasted_iota(jnp.int32, (128, GROUP), 1)
    wl_cat = jnp.concatenate([wl] * GROUP, axis=0)
    wr_cat = jnp.concatenate([wr] * GROUP, axis=0)
    kl = jnp.where(i // D_H == j, wl_cat, 0.0)
    kr = jnp.where(i // D_H == j, wr_cat, 0.0)
    out_ref[...] = (jnp.dot(agg2, kl, preferred_element_type=jnp.float32)
                    + jnp.dot(h, kr, preferred_element_type=jnp.float32) + c)


def kernel(x, edge_index, W1_l, W1_r, b1, W2_l, W2_r, b2, fc_w, fc_b):
    ei3 = edge_index.reshape(2, ROWS, CHUNK).transpose(1, 0, 2)
    zeros = jnp.zeros((NPAD, D_H), jnp.float32)
    onesc = jnp.tile(
        jnp.array([1.0] + [0.0] * (D_H - 1), jnp.float32)[None, :],
        (CHUNK, 1))
    x2 = x.reshape(NP, GROUP * 128)

    y_p, r_p = pl.pallas_call(
        _dense1_body,
        out_shape=(jax.ShapeDtypeStruct((NP, 128), jnp.float32),
                   jax.ShapeDtypeStruct((NP, 128), jnp.float32)),
    )(x2, W1_l, W1_r)

    aggp, degp = _make_edge_pass(True)(
        zeros, onesc, y_p.reshape(N, D_H), ei3)

    h_p, dinv_p = pl.pallas_call(
        _mid_body,
        out_shape=(jax.ShapeDtypeStruct((NP, 128), jnp.float32),
                   jax.ShapeDtypeStruct((NP, 128), jnp.float32)),
    )(aggp.reshape(2, NPP, 128), degp.reshape(2, NPP, 128), r_p,
      b1.reshape(1, D_H))

    agg2p = _make_edge_pass(False)(zeros, h_p.reshape(N, D_H), ei3)

    out_p = pl.pallas_call(
        _out_body,
        out_shape=jax.ShapeDtypeStruct((NP, GROUP), jnp.float32),
    )(agg2p.reshape(2, NPP, 128), dinv_p, h_p, W2_l, W2_r,
      b2.reshape(1, 8), fc_w, fc_b.reshape(1, 1))
    return out_p.reshape(N, 1)


# speedup vs baseline: 1.3847x; 1.0440x over previous
import functools

import jax
import jax.numpy as jnp
from jax import lax
from jax.experimental import pallas as pl
from jax.experimental.pallas import tpu as pltpu
from jax.experimental.pallas import tpu_sc as plsc

N = 10000
E = 320000
D_H = 16

NW = 32
CHUNK = 128
ROWS = E // CHUNK
CPW = ROWS // NW
EXTRA = ROWS - NW * CPW
NPAD = 10112
RPT = NPAD // 16
NITER = CPW // 2


def _edge_pass_body(with_deg, *refs):
    if with_deg:
        (zeros_hbm, ones_hbm, table_hbm, ei_hbm,
         agg_out, deg_out,
         ei, buf_a, buf_b, ones_v, agg_s, deg_s, table_s,
         gsem_a, gsem_b, ssem_a, ssem_b, dsem_a, dsem_b) = refs
    else:
        (zeros_hbm, table_hbm, ei_hbm,
         agg_out,
         ei, buf_a, buf_b, agg_s, table_s,
         gsem_a, gsem_b, ssem_a, ssem_b) = refs

    cid = lax.axis_index("c")
    sid = lax.axis_index("s")
    w = cid * 16 + sid

    pltpu.sync_copy(ei_hbm.at[pl.ds(w * CPW, CPW)], ei.at[pl.ds(0, CPW)])

    @pl.when(w < EXTRA)
    def _():
        pltpu.sync_copy(ei_hbm.at[pl.ds(NW * CPW + w, 1)],
                        ei.at[pl.ds(CPW, 1)])

    z = sid * RPT
    zt = sid * (N // 16)
    pltpu.sync_copy(zeros_hbm.at[pl.ds(z, RPT)], agg_s.at[pl.ds(z, RPT)])
    pltpu.sync_copy(table_hbm.at[pl.ds(zt, N // 16)],
                    table_s.at[pl.ds(zt, N // 16)])
    if with_deg:
        pltpu.sync_copy(zeros_hbm.at[pl.ds(z, RPT)], deg_s.at[pl.ds(z, RPT)])
        pltpu.sync_copy(ones_hbm, ones_v)
    plsc.subcore_barrier()

    def gather(j, buf, sem):
        return pltpu.make_async_copy(table_s.at[ei.at[j, 0]], buf, sem)

    def scatter(j, buf, sem):
        return pltpu.make_async_copy(buf, agg_s.at[ei.at[j, 1]], sem)

    def deg_scatter(j, sem):
        return pltpu.make_async_copy(ones_v, deg_s.at[ei.at[j, 1]], sem)

    def scatter_start(j, buf, sem, dsem):
        scatter(j, buf, sem).start(add=True)
        if with_deg:
            deg_scatter(j, dsem).start(add=True)

    def scatter_wait(j, buf, sem, dsem):
        scatter(j, buf, sem).wait()
        if with_deg:
            deg_scatter(j, dsem).wait()

    gather(0, buf_a, gsem_a).start()
    gather(1, buf_b, gsem_b).start()

    def loop_body(i, carry):
        j0 = 2 * i
        j1 = j0 + 1
        gather(j0, buf_a, gsem_a).wait()
        scatter_start(j0, buf_a, ssem_a, dsem_a if with_deg else None)
        gather(j1, buf_b, gsem_b).wait()
        scatter_start(j1, buf_b, ssem_b, dsem_b if with_deg else None)

        @pl.when(i < NITER - 1)
        def _():
            scatter_wait(j0, buf_a, ssem_a, dsem_a if with_deg else None)
            gather(j0 + 2, buf_a, gsem_a).start()
            scatter_wait(j1, buf_b, ssem_b, dsem_b if with_deg else None)
            gather(j1 + 2, buf_b, gsem_b).start()

        return carry

    lax.fori_loop(0, NITER, loop_body, 0)
    scatter_wait(2 * NITER - 2, buf_a, ssem_a, dsem_a if with_deg else None)
    scatter_wait(2 * NITER - 1, buf_b, ssem_b, dsem_b if with_deg else None)

    @pl.when(w < EXTRA)
    def _():
        gather(CPW, buf_a, gsem_a).start()
        gather(CPW, buf_a, gsem_a).wait()
        scatter(CPW, buf_a, ssem_a).start(add=True)
        if with_deg:
            deg_scatter(CPW, dsem_a).start(add=True)
        scatter(CPW, buf_a, ssem_a).wait()
        if with_deg:
            deg_scatter(CPW, dsem_a).wait()

    plsc.subcore_barrier()

    pltpu.sync_copy(agg_s.at[pl.ds(z, RPT)], agg_out.at[cid, pl.ds(z, RPT)])
    if with_deg:
        pltpu.sync_copy(deg_s.at[pl.ds(z, RPT)],
                        deg_out.at[cid, pl.ds(z, RPT)])


def _make_edge_pass(with_deg):
    mesh = plsc.VectorSubcoreMesh(core_axis_name="c", subcore_axis_name="s")
    out_type = [jax.ShapeDtypeStruct((2, NPAD, D_H), jnp.float32)]
    if with_deg:
        out_type.append(jax.ShapeDtypeStruct((2, NPAD, D_H), jnp.float32))
    scratch = [
        pltpu.VMEM((CPW + 1, 2, CHUNK), jnp.int32),
        pltpu.VMEM((CHUNK, D_H), jnp.float32),
        pltpu.VMEM((CHUNK, D_H), jnp.float32),
    ]
    if with_deg:
        scratch.append(pltpu.VMEM((CHUNK, D_H), jnp.float32))
    scratch.append(pltpu.VMEM_SHARED((NPAD, D_H), jnp.float32))
    if with_deg:
        scratch.append(pltpu.VMEM_SHARED((NPAD, D_H), jnp.float32))
    scratch.append(pltpu.VMEM_SHARED((N, D_H), jnp.float32))
    scratch += [pltpu.SemaphoreType.DMA] * (6 if with_deg else 4)

    return pl.kernel(
        functools.partial(_edge_pass_body, with_deg),
        mesh=mesh,
        out_type=tuple(out_type) if with_deg else out_type[0],
        scratch_types=scratch,
        compiler_params=pltpu.CompilerParams(use_tc_tiling_on_sc=False),
    )


NP = N * D_H // 128
NPP = NPAD * D_H // 128
GROUP = 128 // D_H


def _dense1_body(x_ref, wl_ref, wr_ref, y_ref, r_ref):
    x = x_ref[...].reshape(NP, GROUP * 128)
    i = lax.broadcasted_iota(jnp.int32, (GROUP * 128, 128), 0)
    j = lax.broadcasted_iota(jnp.int32, (GROUP * 128, 128), 1)
    m = (i // 128) == (j // D_H)

    def big(w):
        wcat = jnp.concatenate([w] * GROUP, axis=0)
        wt = jnp.concatenate([wcat] * GROUP, axis=1)
        return jnp.where(m, wt, 0.0)

    y_ref[...] = jnp.dot(x, big(wl_ref[...]),
                         preferred_element_type=jnp.float32)
    r_ref[...] = jnp.dot(x, big(wr_ref[...]),
                         preferred_element_type=jnp.float32)


def _mid_body(aggp_ref, degp_ref, r_ref, b1_ref, h_ref, dinv_ref):
    agg = aggp_ref[0, :NP] + aggp_ref[1, :NP]
    deg = degp_ref[0, :NP] + degp_ref[1, :NP]
    i = lax.broadcasted_iota(jnp.int32, (128, 128), 0)
    j = lax.broadcasted_iota(jnp.int32, (128, 128), 1)
    b = jnp.where((i % D_H == 0) & (i // D_H == j // D_H), 1.0, 0.0)
    degb = jnp.dot(deg, b, preferred_element_type=jnp.float32)
    dinv = 1.0 / jnp.maximum(degb, 1.0)
    b1row = jnp.concatenate([b1_ref[...]] * GROUP, axis=1)
    h_ref[...] = jnp.maximum(agg * dinv + b1row + r_ref[...], 0.0)
    dinv_ref[...] = dinv


def _out_body(agg2p_ref, dinv_ref, h_ref, w2l_ref, w2r_ref, b2_ref,
              fcw_ref, fcb_ref, out_ref):
    h = h_ref[...]
    agg2 = (agg2p_ref[0, :NP] + agg2p_ref[1, :NP]) * dinv_ref[...]
    fcw = fcw_ref[...]
    wl = jnp.dot(w2l_ref[...], fcw, preferred_element_type=jnp.float32)
    wr = jnp.dot(w2r_ref[...], fcw, preferred_element_type=jnp.float32)
    c = jnp.dot(b2_ref[...], fcw, preferred_element_type=jnp.float32) + fcb_ref[...]
    i = lax.broadcasted_iota(jnp.int32, (128, GROUP), 0)
    j = lax.broadcasted_iota(jnp.int32, (128, GROUP), 1)
    wl_cat = jnp.concatenate([wl] * GROUP, axis=0)
    wr_cat = jnp.concatenate([wr] * GROUP, axis=0)
    kl = jnp.where(i // D_H == j, wl_cat, 0.0)
    kr = jnp.where(i // D_H == j, wr_cat, 0.0)
    out_ref[...] = (jnp.dot(agg2, kl, preferred_element_type=jnp.float32)
                    + jnp.dot(h, kr, preferred_element_type=jnp.float32) + c)


def kernel(x, edge_index, W1_l, W1_r, b1, W2_l, W2_r, b2, fc_w, fc_b):
    ei3 = edge_index.reshape(2, ROWS, CHUNK).transpose(1, 0, 2)
    zeros = jnp.zeros((NPAD, D_H), jnp.float32)
    onesc = jnp.tile(
        jnp.array([1.0] + [0.0] * (D_H - 1), jnp.float32)[None, :],
        (CHUNK, 1))
    x3 = x.reshape(NP, GROUP, 128)

    y_p, r_p = pl.pallas_call(
        _dense1_body,
        out_shape=(jax.ShapeDtypeStruct((NP, 128), jnp.float32),
                   jax.ShapeDtypeStruct((NP, 128), jnp.float32)),
    )(x3, W1_l, W1_r)

    aggp, degp = _make_edge_pass(True)(
        zeros, onesc, y_p.reshape(N, D_H), ei3)

    h_p, dinv_p = pl.pallas_call(
        _mid_body,
        out_shape=(jax.ShapeDtypeStruct((NP, 128), jnp.float32),
                   jax.ShapeDtypeStruct((NP, 128), jnp.float32)),
    )(aggp.reshape(2, NPP, 128), degp.reshape(2, NPP, 128), r_p,
      b1.reshape(1, D_H))

    agg2p = _make_edge_pass(False)(zeros, h_p.reshape(N, D_H), ei3)

    out_p = pl.pallas_call(
        _out_body,
        out_shape=jax.ShapeDtypeStruct((NP, GROUP), jnp.float32),
    )(agg2p.reshape(2, NPP, 128), dinv_p, h_p, W2_l, W2_r,
      b2.reshape(1, 8), fc_w, fc_b.reshape(1, 1))
    return out_p.reshape(N, 1)


# speedup vs baseline: 1.4232x; 1.0278x over previous
import functools

import jax
import jax.numpy as jnp
from jax import lax
from jax.experimental import pallas as pl
from jax.experimental.pallas import tpu as pltpu
from jax.experimental.pallas import tpu_sc as plsc

N = 10000
E = 320000
D_H = 16

NW = 32
CHUNK = 128
ROWS = E // CHUNK
CPW = ROWS // NW
EXTRA = ROWS - NW * CPW
NPAD = 10112
RPT = NPAD // 16
NITER = CPW // 2


def _edge_pass_body(with_deg, *refs):
    if with_deg:
        (zeros_hbm, ones_hbm, table_hbm, ei_hbm,
         agg_out, deg_out,
         ei, buf_a, buf_b, ones_v, agg_s, deg_s, table_s,
         gsem_a, gsem_b, ssem_a, ssem_b, dsem_a, dsem_b) = refs
    else:
        (zeros_hbm, table_hbm, ei_hbm,
         agg_out,
         ei, buf_a, buf_b, agg_s, table_s,
         gsem_a, gsem_b, ssem_a, ssem_b) = refs

    cid = lax.axis_index("c")
    sid = lax.axis_index("s")
    w = cid * 16 + sid

    pltpu.sync_copy(ei_hbm.at[pl.ds(w * CPW, CPW)], ei.at[pl.ds(0, CPW)])

    @pl.when(w < EXTRA)
    def _():
        pltpu.sync_copy(ei_hbm.at[pl.ds(NW * CPW + w, 1)],
                        ei.at[pl.ds(CPW, 1)])

    z = sid * RPT
    zt = sid * (N // 16)
    pltpu.sync_copy(zeros_hbm.at[pl.ds(z, RPT)], agg_s.at[pl.ds(z, RPT)])
    pltpu.sync_copy(table_hbm.at[pl.ds(zt, N // 16)],
                    table_s.at[pl.ds(zt, N // 16)])
    if with_deg:
        pltpu.sync_copy(zeros_hbm.at[pl.ds(z, RPT)], deg_s.at[pl.ds(z, RPT)])
        pltpu.sync_copy(ones_hbm, ones_v)
    plsc.subcore_barrier()

    def gather(j, buf, sem):
        return pltpu.make_async_copy(table_s.at[ei.at[j, 0]], buf, sem)

    def scatter(j, buf, sem):
        return pltpu.make_async_copy(buf, agg_s.at[ei.at[j, 1]], sem)

    def deg_scatter(j, sem):
        return pltpu.make_async_copy(ones_v, deg_s.at[ei.at[j, 1]], sem)

    def scatter_start(j, buf, sem, dsem):
        scatter(j, buf, sem).start(add=True)
        if with_deg:
            deg_scatter(j, dsem).start(add=True)

    def scatter_wait(j, buf, sem, dsem):
        scatter(j, buf, sem).wait()
        if with_deg:
            deg_scatter(j, dsem).wait()

    gather(0, buf_a, gsem_a).start()
    gather(1, buf_b, gsem_b).start()

    def loop_body(i, carry):
        j0 = 2 * i
        j1 = j0 + 1
        gather(j0, buf_a, gsem_a).wait()
        scatter_start(j0, buf_a, ssem_a, dsem_a if with_deg else None)
        gather(j1, buf_b, gsem_b).wait()
        scatter_start(j1, buf_b, ssem_b, dsem_b if with_deg else None)

        @pl.when(i < NITER - 1)
        def _():
            scatter_wait(j0, buf_a, ssem_a, dsem_a if with_deg else None)
            gather(j0 + 2, buf_a, gsem_a).start()
            scatter_wait(j1, buf_b, ssem_b, dsem_b if with_deg else None)
            gather(j1 + 2, buf_b, gsem_b).start()

        return carry

    lax.fori_loop(0, NITER, loop_body, 0)
    scatter_wait(2 * NITER - 2, buf_a, ssem_a, dsem_a if with_deg else None)
    scatter_wait(2 * NITER - 1, buf_b, ssem_b, dsem_b if with_deg else None)

    @pl.when(w < EXTRA)
    def _():
        gather(CPW, buf_a, gsem_a).start()
        gather(CPW, buf_a, gsem_a).wait()
        scatter(CPW, buf_a, ssem_a).start(add=True)
        if with_deg:
            deg_scatter(CPW, dsem_a).start(add=True)
        scatter(CPW, buf_a, ssem_a).wait()
        if with_deg:
            deg_scatter(CPW, dsem_a).wait()

    plsc.subcore_barrier()

    pltpu.sync_copy(agg_s.at[pl.ds(z, RPT)], agg_out.at[cid, pl.ds(z, RPT)])
    if with_deg:
        pltpu.sync_copy(deg_s.at[pl.ds(z, RPT)],
                        deg_out.at[cid, pl.ds(z, RPT)])


def _make_edge_pass(with_deg):
    mesh = plsc.VectorSubcoreMesh(core_axis_name="c", subcore_axis_name="s")
    out_type = [jax.ShapeDtypeStruct((2, NPAD, D_H), jnp.float32)]
    if with_deg:
        out_type.append(jax.ShapeDtypeStruct((2, NPAD, D_H), jnp.float32))
    scratch = [
        pltpu.VMEM((CPW + 1, 2, CHUNK), jnp.int32),
        pltpu.VMEM((CHUNK, D_H), jnp.float32),
        pltpu.VMEM((CHUNK, D_H), jnp.float32),
    ]
    if with_deg:
        scratch.append(pltpu.VMEM((CHUNK, D_H), jnp.float32))
    scratch.append(pltpu.VMEM_SHARED((NPAD, D_H), jnp.float32))
    if with_deg:
        scratch.append(pltpu.VMEM_SHARED((NPAD, D_H), jnp.float32))
    scratch.append(pltpu.VMEM_SHARED((N, D_H), jnp.float32))
    scratch += [pltpu.SemaphoreType.DMA] * (6 if with_deg else 4)

    return pl.kernel(
        functools.partial(_edge_pass_body, with_deg),
        mesh=mesh,
        out_type=tuple(out_type) if with_deg else out_type[0],
        scratch_types=scratch,
        compiler_params=pltpu.CompilerParams(use_tc_tiling_on_sc=False),
    )


NP = N * D_H // 128
NPP = NPAD * D_H // 128
GROUP = 128 // D_H


def _dense1_body(x_ref, wlt_ref, wrt_ref, y_ref, r_ref):
    x = x_ref[...].reshape(NP, GROUP * 128)
    i = lax.broadcasted_iota(jnp.int32, (GROUP * 128, 128), 0)
    j = lax.broadcasted_iota(jnp.int32, (GROUP * 128, 128), 1)
    m = (i // 128) == (j // D_H)

    def big(w):
        wcat = jnp.concatenate([w] * GROUP, axis=0)
        wt = jnp.concatenate([wcat] * GROUP, axis=1)
        return jnp.where(m, wt, 0.0)

    y_ref[...] = jnp.dot(x, big(wlt_ref[...].T),
                         preferred_element_type=jnp.float32)
    r_ref[...] = jnp.dot(x, big(wrt_ref[...].T),
                         preferred_element_type=jnp.float32)


def _mid_body(aggp_ref, degp_ref, r_ref, b1_ref, h_ref, dinv_ref):
    agg = aggp_ref[0, :NP] + aggp_ref[1, :NP]
    deg = degp_ref[0, :NP] + degp_ref[1, :NP]
    i = lax.broadcasted_iota(jnp.int32, (128, 128), 0)
    j = lax.broadcasted_iota(jnp.int32, (128, 128), 1)
    b = jnp.where((i % D_H == 0) & (i // D_H == j // D_H), 1.0, 0.0)
    degb = jnp.dot(deg, b, preferred_element_type=jnp.float32)
    dinv = 1.0 / jnp.maximum(degb, 1.0)
    b1row = jnp.concatenate([b1_ref[...]] * GROUP, axis=1)
    h_ref[...] = jnp.maximum(agg * dinv + b1row + r_ref[...], 0.0)
    dinv_ref[...] = dinv


def _out_body(agg2p_ref, dinv_ref, h_ref, w2lt_ref, w2rt_ref, b2_ref,
              fcw_ref, fcb_ref, out_ref):
    h = h_ref[...]
    agg2 = (agg2p_ref[0, :NP] + agg2p_ref[1, :NP]) * dinv_ref[...]
    fcw = fcw_ref[...]
    wl = jnp.dot(w2lt_ref[...].T, fcw, preferred_element_type=jnp.float32)
    wr = jnp.dot(w2rt_ref[...].T, fcw, preferred_element_type=jnp.float32)
    c = jnp.dot(b2_ref[...], fcw, preferred_element_type=jnp.float32) + fcb_ref[...]
    i = lax.broadcasted_iota(jnp.int32, (128, GROUP), 0)
    j = lax.broadcasted_iota(jnp.int32, (128, GROUP), 1)
    wl_cat = jnp.concatenate([wl] * GROUP, axis=0)
    wr_cat = jnp.concatenate([wr] * GROUP, axis=0)
    kl = jnp.where(i // D_H == j, wl_cat, 0.0)
    kr = jnp.where(i // D_H == j, wr_cat, 0.0)
    out_ref[...] = (jnp.dot(agg2, kl, preferred_element_type=jnp.float32)
                    + jnp.dot(h, kr, preferred_element_type=jnp.float32) + c)


def kernel(x, edge_index, W1_l, W1_r, b1, W2_l, W2_r, b2, fc_w, fc_b):
    ei3 = edge_index.reshape(2, ROWS, CHUNK).transpose(1, 0, 2)
    zeros = jnp.zeros((NPAD, D_H), jnp.float32)
    onesc = jnp.tile(
        jnp.array([1.0] + [0.0] * (D_H - 1), jnp.float32)[None, :],
        (CHUNK, 1))
    x3 = x.reshape(NP, GROUP, 128)

    y_p, r_p = pl.pallas_call(
        _dense1_body,
        out_shape=(jax.ShapeDtypeStruct((NP, 128), jnp.float32),
                   jax.ShapeDtypeStruct((NP, 128), jnp.float32)),
    )(x3, W1_l.T, W1_r.T)

    aggp, degp = _make_edge_pass(True)(
        zeros, onesc, y_p.reshape(N, D_H), ei3)

    h_p, dinv_p = pl.pallas_call(
        _mid_body,
        out_shape=(jax.ShapeDtypeStruct((NP, 128), jnp.float32),
                   jax.ShapeDtypeStruct((NP, 128), jnp.float32)),
    )(aggp.reshape(2, NPP, 128), degp.reshape(2, NPP, 128), r_p,
      b1.reshape(1, D_H))

    agg2p = _make_edge_pass(False)(zeros, h_p.reshape(N, D_H), ei3)

    out_p = pl.pallas_call(
        _out_body,
        out_shape=jax.ShapeDtypeStruct((NP, GROUP), jnp.float32),
    )(agg2p.reshape(2, NPP, 128), dinv_p, h_p, W2_l.T, W2_r.T,
      b2.reshape(1, 8), fc_w, fc_b.reshape(1, 1))
    return out_p.reshape(N, 1)
